# Initial kernel scaffold; baseline (speedup 1.0000x reference)
import jax
import jax.numpy as jnp
from jax.experimental import pallas as pl


def kernel(inputs, loc_info, sparse_idx, geodesic, angle_ratio, node_embeddings, W_feat, b_feat, Wk1, bk1, Wk2, bk2, W_conv1, W_conv2, W_out, b_out):
    raise NotImplementedError("write your pallas kernel here")


# speedup vs baseline: 24.4359x; 24.4359x over previous
import functools

import jax
import jax.numpy as jnp
from jax import lax
from jax.experimental import pallas as pl
from jax.experimental.pallas import tpu as pltpu
from jax.experimental.pallas import tpu_sc as plsc

N = 10000
E = 160000
F32 = jnp.float32

NC = 2
NT = 16
KC = 128
NCHUNK = 80
EPAD = NT * NCHUNK * KC
CH = 128
NP = 10240
RPT = NP // NT

_sds = jax.ShapeDtypeStruct

_mesh = plsc.VectorSubcoreMesh(
    core_axis_name="c", subcore_axis_name="s", num_cores=NC, num_subcores=NT)


def _dloc_body(locT, srcr, dstr, dx_out, dy_out, locx, locy, src_v, dst_v,
               dxb, dyb):
    c = lax.axis_index("c")
    s = lax.axis_index("s")
    pltpu.sync_copy(locT.at[0], locx)
    pltpu.sync_copy(locT.at[1], locy)
    half = NCHUNK // NC
    pltpu.sync_copy(srcr.at[s, pl.ds(c * half, half)], src_v)
    pltpu.sync_copy(dstr.at[s, pl.ds(c * half, half)], dst_v)

    def row(i, carry):
        for q in range(KC // 16):
            sl = pl.ds(q * 16, 16)
            si = src_v[i, sl]
            di = dst_v[i, sl]
            dxb[i, sl] = plsc.load_gather(locx, [di]) - plsc.load_gather(locx, [si])
            dyb[i, sl] = plsc.load_gather(locy, [di]) - plsc.load_gather(locy, [si])
        return carry

    lax.fori_loop(0, half, row, 0)
    pltpu.sync_copy(dxb, dx_out.at[s, pl.ds(c * half, half)])
    pltpu.sync_copy(dyb, dy_out.at[s, pl.ds(c * half, half)])


_dloc = functools.partial(
    pl.kernel,
    out_type=[_sds((NT, NCHUNK, KC), F32), _sds((NT, NCHUNK, KC), F32)],
    mesh=_mesh,
    compiler_params=pltpu.CompilerParams(needs_layout_passes=False),
    scratch_types=[
        pltpu.VMEM((10240,), F32),
        pltpu.VMEM((10240,), F32),
        pltpu.VMEM((NCHUNK // NC, KC), jnp.int32),
        pltpu.VMEM((NCHUNK // NC, KC), jnp.int32),
        pltpu.VMEM((NCHUNK // NC, KC), F32),
        pltpu.VMEM((NCHUNK // NC, KC), F32),
    ],
)(_dloc_body)


NQ = 5
QCH = NCHUNK // NQ


def _agg(table, acc, srcr2, dstr, wr, c, s, src_v, dst_v, w_v, rows, gsem):
    def quarter(q4, carry):
        off = pl.ds(q4 * QCH, QCH)
        pltpu.sync_copy(srcr2.at[c, s, off], src_v)
        pltpu.sync_copy(dstr.at[s, off], dst_v)
        pltpu.sync_copy(wr.at[s, off], w_v)

        def chunk(j, c1):
            pltpu.async_copy(table.at[src_v.at[j]], rows, gsem).wait()

            def egroup(g, c2):
                w16 = w_v[j, pl.ds(g * 16, 16)]
                for i in range(16):
                    wvec = jnp.broadcast_to(w16[i], (16,))
                    e = g * 16 + i
                    for q in range(CH // 16):
                        sl = pl.ds(q * 16, 16)
                        rows[e, sl] = rows[e, sl] * wvec
                return c2

            lax.fori_loop(0, KC // 16, egroup, 0)
            pltpu.sync_copy(rows, acc.at[dst_v.at[j]], add=True)
            return c1

        lax.fori_loop(0, QCH, chunk, 0)
        return carry

    lax.fori_loop(0, NQ, quarter, 0)


def _block_body(y0r, y1r, y2r, srcr2, dstr, wr, tmid, hout,
                acc, src_v, dst_v, w_v, rows, gsem):
    c = lax.axis_index("c")
    s = lax.axis_index("s")
    srow = s * RPT
    hbase = c * NP + srow
    pltpu.sync_copy(y1r.at[pl.ds(hbase, RPT)], acc.at[pl.ds(srow, RPT)])
    plsc.subcore_barrier()
    _agg(y2r, acc, srcr2, dstr, wr, c, s, src_v, dst_v, w_v, rows, gsem)
    plsc.subcore_barrier()
    pltpu.sync_copy(acc.at[pl.ds(srow, RPT)], tmid.at[pl.ds(hbase, RPT)])
    pltpu.sync_copy(y0r.at[pl.ds(hbase, RPT)], acc.at[pl.ds(srow, RPT)])
    plsc.subcore_barrier()
    _agg(tmid, acc, srcr2, dstr, wr, c, s, src_v, dst_v, w_v, rows, gsem)
    plsc.subcore_barrier()
    piece = RPT // 5
    for t in range(5):
        pltpu.sync_copy(acc.at[pl.ds(srow + t * piece, piece)], rows)

        def rrow(i, carry):
            for q in range(CH // 16):
                sl = pl.ds(q * 16, 16)
                rows[i, sl] = jnp.maximum(rows[i, sl], 0.0)
            return carry

        lax.fori_loop(0, piece, rrow, 0)
        pltpu.sync_copy(rows, hout.at[pl.ds(hbase + t * piece, piece)])


_blockk = functools.partial(
    pl.kernel,
    out_type=[_sds((NC * NP, CH), F32), _sds((NC * NP, CH), F32)],
    mesh=_mesh,
    compiler_params=pltpu.CompilerParams(needs_layout_passes=False),
    scratch_types=[
        pltpu.VMEM_SHARED((NP, CH), F32),
        pltpu.VMEM((QCH, KC), jnp.int32),
        pltpu.VMEM((QCH, KC), jnp.int32),
        pltpu.VMEM((QCH, KC), F32),
        pltpu.VMEM((KC, CH), F32),
        pltpu.SemaphoreType.DMA,
    ],
)(_block_body)


def _mlp_block(x_ref, w1_ref, b1_ref, w2_ref, b2_ref, o_ref):
    h = jnp.tanh(
        jnp.dot(x_ref[...], w1_ref[...], preferred_element_type=F32)
        + b1_ref[...])
    o_ref[...] = jnp.dot(h, w2_ref[...], preferred_element_type=F32) + b2_ref[...]


def _mlp(ker_in, Wk1, bk1, Wk2, bk2):
    blk = 8192
    grid = EPAD // blk
    return pl.pallas_call(
        _mlp_block,
        grid=(grid,),
        in_specs=[
            pl.BlockSpec((blk, 4), lambda i: (i, 0)),
            pl.BlockSpec((4, 16), lambda i: (0, 0)),
            pl.BlockSpec((1, 16), lambda i: (0, 0)),
            pl.BlockSpec((16, 1), lambda i: (0, 0)),
            pl.BlockSpec((1, 1), lambda i: (0, 0)),
        ],
        out_specs=pl.BlockSpec((blk, 1), lambda i: (i, 0)),
        out_shape=_sds((EPAD, 1), F32),
    )(ker_in, Wk1, bk1, Wk2, bk2)


def _proj_block(x_ref, wf_ref, bf_ref, wc_ref, o_ref):
    wc = wc_ref[...]
    wcat = jnp.concatenate([wc[0:18], wc[18:36], wc[36:54]], axis=1)
    m = jnp.concatenate(
        [jnp.dot(wf_ref[...], wcat[0:8], preferred_element_type=F32)
         + wcat[16:18],
         wcat[8:16]], axis=0)
    c0 = jnp.dot(bf_ref[...], wcat[0:8], preferred_element_type=F32)
    o_ref[...] = jnp.dot(x_ref[...], m, preferred_element_type=F32) + c0


def _proj(xaug, W_feat, b_feat, W_conv1):
    blk = 8000
    grid = (N * 16) // blk
    return pl.pallas_call(
        _proj_block,
        grid=(grid,),
        in_specs=[
            pl.BlockSpec((blk, 10), lambda i: (i, 0)),
            pl.BlockSpec((2, 8), lambda i: (0, 0)),
            pl.BlockSpec((1, 8), lambda i: (0, 0)),
            pl.BlockSpec((54, 16), lambda i: (0, 0)),
        ],
        out_specs=pl.BlockSpec((blk, 48), lambda i: (i, 0)),
        out_shape=_sds((N * 16, 48), F32),
    )(xaug, W_feat, b_feat, W_conv1)


def _mid_block(x_ref, wc2_ref, o_ref):
    wc2 = wc2_ref[...]
    vcat = jnp.concatenate([wc2[0:16], wc2[16:32], wc2[32:48]], axis=1)
    o_ref[...] = jnp.dot(x_ref[...], vcat, preferred_element_type=F32)


def _mid(hT, W_conv2):
    blk = 8000
    grid = (N * 16) // blk
    return pl.pallas_call(
        _mid_block,
        grid=(grid,),
        in_specs=[
            pl.BlockSpec((blk, 16), lambda i: (i, 0)),
            pl.BlockSpec((48, 16), lambda i: (0, 0)),
        ],
        out_specs=pl.BlockSpec((blk, 48), lambda i: (i, 0)),
        out_shape=_sds((N * 16, 48), F32),
    )(hT, W_conv2)


def _final_block(x_ref, wo_ref, bo_ref, o_ref):
    wo = wo_ref[...]
    wbig = jnp.concatenate([wo] * 8, axis=0) * (1.0 / 8.0)
    o_ref[...] = (
        jnp.dot(x_ref[...], wbig, preferred_element_type=F32) + bo_ref[...])


def _final(h2, W_out, b_out):
    blk = 2000
    grid = (NC * N) // blk
    return pl.pallas_call(
        _final_block,
        grid=(grid,),
        in_specs=[
            pl.BlockSpec((blk, CH), lambda i: (i, 0)),
            pl.BlockSpec((16, 8), lambda i: (0, 0)),
            pl.BlockSpec((1, 8), lambda i: (0, 0)),
        ],
        out_specs=pl.BlockSpec((blk, 8), lambda i: (i, 0)),
        out_shape=_sds((NC * N, 8), F32),
    )(h2, W_out, b_out)


def kernel(inputs, loc_info, sparse_idx, geodesic, angle_ratio,
           node_embeddings, W_feat, b_feat, Wk1, bk1, Wk2, bk2,
           W_conv1, W_conv2, W_out, b_out):
    dst = sparse_idx[0]
    src = sparse_idx[1]
    pad = EPAD - E
    zi = jnp.zeros((pad,), jnp.int32)
    src_p = jnp.concatenate([src, zi]).reshape(NT, NCHUNK, KC)
    dst_p = jnp.concatenate([dst, zi]).reshape(NT, NCHUNK, KC)
    srcr2 = jnp.stack([src_p, src_p + NP])

    locT = jnp.concatenate([loc_info.T, jnp.zeros((2, 10240 - N), F32)], axis=1)
    dx, dy = _dloc(locT, src_p, dst_p)
    zf = jnp.zeros((pad,), F32)
    ker_in = jnp.stack([
        dx.reshape(-1), dy.reshape(-1),
        jnp.concatenate([geodesic, zf]),
        jnp.concatenate([angle_ratio, zf])], axis=-1)
    w_mlp = _mlp(ker_in, Wk1, bk1.reshape(1, 16), Wk2, bk2.reshape(1, 1))
    wr = jnp.concatenate([w_mlp[:E, 0], zf]).reshape(NT, NCHUNK, KC)

    xflat = jnp.transpose(inputs, (2, 0, 1, 3)).reshape(N * 16, 2)
    nodeb = jnp.broadcast_to(
        node_embeddings[:, None, :], (N, 16, 8)).reshape(N * 16, 8)
    xaug = jnp.concatenate([xflat, nodeb], axis=1)
    y_all = _proj(xaug, W_feat, b_feat.reshape(1, 8), W_conv1)

    def to_tabs(flat):
        t = flat.reshape(N, 2, 8, 3, 16).transpose(3, 1, 0, 2, 4).reshape(
            3, NC, N, CH)
        t = jnp.pad(t, ((0, 0), (0, 0), (0, NP - N), (0, 0)))
        return t.reshape(3, NC * NP, CH)

    ytab = to_tabs(y_all)
    _, h1 = _blockk(ytab[0], ytab[1], ytab[2], srcr2, dst_p, wr)

    hT = h1.reshape(2, NP, 8, 16)[:, :N].transpose(1, 0, 2, 3).reshape(
        N * 16, 16)
    z_all = _mid(hT, W_conv2)
    ztab = to_tabs(z_all)
    _, h2 = _blockk(ztab[0], ztab[1], ztab[2], srcr2, dst_p, wr)

    h2s = h2.reshape(2, NP, CH)[:, :N].reshape(2 * N, CH)
    o = _final(h2s, W_out, b_out.reshape(1, 8))
    return o.reshape(2, N, 8).transpose(2, 0, 1)


# speedup vs baseline: 26.6084x; 1.0889x over previous
import functools

import jax
import jax.numpy as jnp
from jax import lax
from jax.experimental import pallas as pl
from jax.experimental.pallas import tpu as pltpu
from jax.experimental.pallas import tpu_sc as plsc

N = 10000
E = 160000
F32 = jnp.float32

NC = 2
NT = 16
KC = 64
NCHUNK = 160
EPAD = NT * NCHUNK * KC
CH = 128
NP = 10240
RPT = NP // NT

_sds = jax.ShapeDtypeStruct

_mesh = plsc.VectorSubcoreMesh(
    core_axis_name="c", subcore_axis_name="s", num_cores=NC, num_subcores=NT)


def _dloc_body(locT, srcr, dstr, dx_out, dy_out, locx, locy, src_v, dst_v,
               dxb, dyb):
    c = lax.axis_index("c")
    s = lax.axis_index("s")
    pltpu.sync_copy(locT.at[0], locx)
    pltpu.sync_copy(locT.at[1], locy)
    half = NCHUNK // NC
    pltpu.sync_copy(srcr.at[s, pl.ds(c * half, half)], src_v)
    pltpu.sync_copy(dstr.at[s, pl.ds(c * half, half)], dst_v)

    def row(i, carry):
        for q in range(KC // 16):
            sl = pl.ds(q * 16, 16)
            si = src_v[i, sl]
            di = dst_v[i, sl]
            dxb[i, sl] = plsc.load_gather(locx, [di]) - plsc.load_gather(locx, [si])
            dyb[i, sl] = plsc.load_gather(locy, [di]) - plsc.load_gather(locy, [si])
        return carry

    lax.fori_loop(0, half, row, 0)
    pltpu.sync_copy(dxb, dx_out.at[s, pl.ds(c * half, half)])
    pltpu.sync_copy(dyb, dy_out.at[s, pl.ds(c * half, half)])


_dloc = functools.partial(
    pl.kernel,
    out_type=[_sds((NT, NCHUNK, KC), F32), _sds((NT, NCHUNK, KC), F32)],
    mesh=_mesh,
    compiler_params=pltpu.CompilerParams(needs_layout_passes=False),
    scratch_types=[
        pltpu.VMEM((10240,), F32),
        pltpu.VMEM((10240,), F32),
        pltpu.VMEM((NCHUNK // NC, KC), jnp.int32),
        pltpu.VMEM((NCHUNK // NC, KC), jnp.int32),
        pltpu.VMEM((NCHUNK // NC, KC), F32),
        pltpu.VMEM((NCHUNK // NC, KC), F32),
    ],
)(_dloc_body)


RFL = 32
NPIECE = NCHUNK // RFL


def _agg(table, acc, srcr2, dstr, wr, c, s, src_v, dst_v, w_v, ra, rb,
         gsem, ssem):
    def scale(buf, j):
        def egroup(g, c2):
            w16 = w_v[j, pl.ds(g * 16, 16)]
            for i in range(16):
                wvec = jnp.broadcast_to(w16[i], (16,))
                e = g * 16 + i
                for q in range(CH // 16):
                    sl = pl.ds(q * 16, 16)
                    buf[e, sl] = buf[e, sl] * wvec
            return c2

        lax.fori_loop(0, KC // 16, egroup, 0)

    def piece(p, carry):
        off = pl.ds(p * RFL, RFL)
        pltpu.sync_copy(srcr2.at[c, s, off], src_v)
        pltpu.sync_copy(dstr.at[s, off], dst_v)
        pltpu.sync_copy(wr.at[s, off], w_v)
        pltpu.async_copy(table.at[src_v.at[0]], ra, gsem)

        def pair(j2, c1):
            j0 = 2 * j2
            j1 = j0 + 1

            @pl.when(j2 > 0)
            def _():
                pltpu.make_async_copy(rb, acc.at[dst_v.at[j0]], ssem).wait()

            pltpu.async_copy(table.at[src_v.at[j1]], rb, gsem)
            pltpu.make_async_copy(table.at[src_v.at[j0]], ra, gsem).wait()
            scale(ra, j0)
            pltpu.async_copy(ra, acc.at[dst_v.at[j0]], ssem, add=True)
            pltpu.make_async_copy(table.at[src_v.at[j1]], rb, gsem).wait()
            scale(rb, j1)
            pltpu.async_copy(rb, acc.at[dst_v.at[j1]], ssem, add=True)
            pltpu.make_async_copy(ra, acc.at[dst_v.at[j0]], ssem).wait()

            @pl.when(j2 < RFL // 2 - 1)
            def _():
                pltpu.async_copy(table.at[src_v.at[j0 + 2]], ra, gsem)

            return c1

        lax.fori_loop(0, RFL // 2, pair, 0)
        pltpu.make_async_copy(rb, acc.at[dst_v.at[RFL - 1]], ssem).wait()
        return carry

    lax.fori_loop(0, NPIECE, piece, 0)


def _block_body(y0r, y1r, y2r, srcr2, dstr, wr, tmid, hout,
                acc, src_v, dst_v, w_v, ra, rb, gsem, ssem):
    c = lax.axis_index("c")
    s = lax.axis_index("s")
    srow = s * RPT
    hbase = c * NP + srow
    pltpu.sync_copy(y1r.at[pl.ds(hbase, RPT)], acc.at[pl.ds(srow, RPT)])
    plsc.subcore_barrier()
    _agg(y2r, acc, srcr2, dstr, wr, c, s, src_v, dst_v, w_v, ra, rb,
         gsem, ssem)
    plsc.subcore_barrier()
    pltpu.sync_copy(acc.at[pl.ds(srow, RPT)], tmid.at[pl.ds(hbase, RPT)])
    pltpu.sync_copy(y0r.at[pl.ds(hbase, RPT)], acc.at[pl.ds(srow, RPT)])
    plsc.subcore_barrier()
    _agg(tmid, acc, srcr2, dstr, wr, c, s, src_v, dst_v, w_v, ra, rb,
         gsem, ssem)
    plsc.subcore_barrier()
    piece = KC
    for t in range(RPT // KC):
        pltpu.sync_copy(acc.at[pl.ds(srow + t * piece, piece)], ra)

        def rrow(i, carry):
            for q in range(CH // 16):
                sl = pl.ds(q * 16, 16)
                ra[i, sl] = jnp.maximum(ra[i, sl], 0.0)
            return carry

        lax.fori_loop(0, piece, rrow, 0)
        pltpu.sync_copy(ra, hout.at[pl.ds(hbase + t * piece, piece)])


_blockk = functools.partial(
    pl.kernel,
    out_type=[_sds((NC * NP, CH), F32), _sds((NC * NP, CH), F32)],
    mesh=_mesh,
    compiler_params=pltpu.CompilerParams(needs_layout_passes=False),
    scratch_types=[
        pltpu.VMEM_SHARED((NP, CH), F32),
        pltpu.VMEM((RFL, KC), jnp.int32),
        pltpu.VMEM((RFL, KC), jnp.int32),
        pltpu.VMEM((RFL, KC), F32),
        pltpu.VMEM((KC, CH), F32),
        pltpu.VMEM((KC, CH), F32),
        pltpu.SemaphoreType.DMA,
        pltpu.SemaphoreType.DMA,
    ],
)(_block_body)


def _mlp_block(x_ref, w1_ref, b1_ref, w2_ref, b2_ref, o_ref):
    h = jnp.tanh(
        jnp.dot(x_ref[...], w1_ref[...], preferred_element_type=F32)
        + b1_ref[...])
    o_ref[...] = jnp.dot(h, w2_ref[...], preferred_element_type=F32) + b2_ref[...]


def _mlp(ker_in, Wk1, bk1, Wk2, bk2):
    blk = 8192
    grid = EPAD // blk
    return pl.pallas_call(
        _mlp_block,
        grid=(grid,),
        in_specs=[
            pl.BlockSpec((blk, 4), lambda i: (i, 0)),
            pl.BlockSpec((4, 16), lambda i: (0, 0)),
            pl.BlockSpec((1, 16), lambda i: (0, 0)),
            pl.BlockSpec((16, 1), lambda i: (0, 0)),
            pl.BlockSpec((1, 1), lambda i: (0, 0)),
        ],
        out_specs=pl.BlockSpec((blk, 1), lambda i: (i, 0)),
        out_shape=_sds((EPAD, 1), F32),
    )(ker_in, Wk1, bk1, Wk2, bk2)


def _proj_block(x_ref, wf_ref, bf_ref, wc_ref, o_ref):
    wc = wc_ref[...]
    wcat = jnp.concatenate([wc[0:18], wc[18:36], wc[36:54]], axis=1)
    m = jnp.concatenate(
        [jnp.dot(wf_ref[...], wcat[0:8], preferred_element_type=F32)
         + wcat[16:18],
         wcat[8:16]], axis=0)
    c0 = jnp.dot(bf_ref[...], wcat[0:8], preferred_element_type=F32)
    o_ref[...] = jnp.dot(x_ref[...], m, preferred_element_type=F32) + c0


def _proj(xaug, W_feat, b_feat, W_conv1):
    blk = 8000
    grid = (N * 16) // blk
    return pl.pallas_call(
        _proj_block,
        grid=(grid,),
        in_specs=[
            pl.BlockSpec((blk, 10), lambda i: (i, 0)),
            pl.BlockSpec((2, 8), lambda i: (0, 0)),
            pl.BlockSpec((1, 8), lambda i: (0, 0)),
            pl.BlockSpec((54, 16), lambda i: (0, 0)),
        ],
        out_specs=pl.BlockSpec((blk, 48), lambda i: (i, 0)),
        out_shape=_sds((N * 16, 48), F32),
    )(xaug, W_feat, b_feat, W_conv1)


def _mid_block(x_ref, wc2_ref, o_ref):
    wc2 = wc2_ref[...]
    vcat = jnp.concatenate([wc2[0:16], wc2[16:32], wc2[32:48]], axis=1)
    o_ref[...] = jnp.dot(x_ref[...], vcat, preferred_element_type=F32)


def _mid(hT, W_conv2):
    blk = 8000
    grid = (N * 16) // blk
    return pl.pallas_call(
        _mid_block,
        grid=(grid,),
        in_specs=[
            pl.BlockSpec((blk, 16), lambda i: (i, 0)),
            pl.BlockSpec((48, 16), lambda i: (0, 0)),
        ],
        out_specs=pl.BlockSpec((blk, 48), lambda i: (i, 0)),
        out_shape=_sds((N * 16, 48), F32),
    )(hT, W_conv2)


def _final_block(x_ref, wo_ref, bo_ref, o_ref):
    wo = wo_ref[...]
    wbig = jnp.concatenate([wo] * 8, axis=0) * (1.0 / 8.0)
    o_ref[...] = (
        jnp.dot(x_ref[...], wbig, preferred_element_type=F32) + bo_ref[...])


def _final(h2, W_out, b_out):
    blk = 2000
    grid = (NC * N) // blk
    return pl.pallas_call(
        _final_block,
        grid=(grid,),
        in_specs=[
            pl.BlockSpec((blk, CH), lambda i: (i, 0)),
            pl.BlockSpec((16, 8), lambda i: (0, 0)),
            pl.BlockSpec((1, 8), lambda i: (0, 0)),
        ],
        out_specs=pl.BlockSpec((blk, 8), lambda i: (i, 0)),
        out_shape=_sds((NC * N, 8), F32),
    )(h2, W_out, b_out)


def kernel(inputs, loc_info, sparse_idx, geodesic, angle_ratio,
           node_embeddings, W_feat, b_feat, Wk1, bk1, Wk2, bk2,
           W_conv1, W_conv2, W_out, b_out):
    dst = sparse_idx[0]
    src = sparse_idx[1]
    pad = EPAD - E
    zi = jnp.zeros((pad,), jnp.int32)
    src_p = jnp.concatenate([src, zi]).reshape(NT, NCHUNK, KC)
    dst_p = jnp.concatenate([dst, zi]).reshape(NT, NCHUNK, KC)
    srcr2 = jnp.stack([src_p, src_p + NP])

    locT = jnp.concatenate([loc_info.T, jnp.zeros((2, 10240 - N), F32)], axis=1)
    dx, dy = _dloc(locT, src_p, dst_p)
    zf = jnp.zeros((pad,), F32)
    ker_in = jnp.stack([
        dx.reshape(-1), dy.reshape(-1),
        jnp.concatenate([geodesic, zf]),
        jnp.concatenate([angle_ratio, zf])], axis=-1)
    w_mlp = _mlp(ker_in, Wk1, bk1.reshape(1, 16), Wk2, bk2.reshape(1, 1))
    wr = jnp.concatenate([w_mlp[:E, 0], zf]).reshape(NT, NCHUNK, KC)

    xflat = jnp.transpose(inputs, (2, 0, 1, 3)).reshape(N * 16, 2)
    nodeb = jnp.broadcast_to(
        node_embeddings[:, None, :], (N, 16, 8)).reshape(N * 16, 8)
    xaug = jnp.concatenate([xflat, nodeb], axis=1)
    y_all = _proj(xaug, W_feat, b_feat.reshape(1, 8), W_conv1)

    def to_tabs(flat):
        t = flat.reshape(N, 2, 8, 3, 16).transpose(3, 1, 0, 2, 4).reshape(
            3, NC, N, CH)
        t = jnp.pad(t, ((0, 0), (0, 0), (0, NP - N), (0, 0)))
        return t.reshape(3, NC * NP, CH)

    ytab = to_tabs(y_all)
    _, h1 = _blockk(ytab[0], ytab[1], ytab[2], srcr2, dst_p, wr)

    hT = h1.reshape(2, NP, 8, 16)[:, :N].transpose(1, 0, 2, 3).reshape(
        N * 16, 16)
    z_all = _mid(hT, W_conv2)
    ztab = to_tabs(z_all)
    _, h2 = _blockk(ztab[0], ztab[1], ztab[2], srcr2, dst_p, wr)

    h2s = h2.reshape(2, NP, CH)[:, :N].reshape(2 * N, CH)
    o = _final(h2s, W_out, b_out.reshape(1, 8))
    return o.reshape(2, N, 8).transpose(2, 0, 1)


# speedup vs baseline: 44.6788x; 1.6791x over previous
import functools

import jax
import jax.numpy as jnp
from jax import lax
from jax.experimental import pallas as pl
from jax.experimental.pallas import tpu as pltpu
from jax.experimental.pallas import tpu_sc as plsc

N = 10000
E = 160000
F32 = jnp.float32

NC = 2
NT = 16
KC = 64
NCHUNK = 160
EPAD = NT * NCHUNK * KC
CH = 128
NP = 10240
RPT = NP // NT

_sds = jax.ShapeDtypeStruct

_mesh = plsc.VectorSubcoreMesh(
    core_axis_name="c", subcore_axis_name="s", num_cores=NC, num_subcores=NT)


def _dloc_body(locT, srcr, dstr, dx_out, dy_out, locx, locy, src_v, dst_v,
               dxb, dyb):
    c = lax.axis_index("c")
    s = lax.axis_index("s")
    pltpu.sync_copy(locT.at[0], locx)
    pltpu.sync_copy(locT.at[1], locy)
    half = NCHUNK // NC
    pltpu.sync_copy(srcr.at[s, pl.ds(c * half, half)], src_v)
    pltpu.sync_copy(dstr.at[s, pl.ds(c * half, half)], dst_v)

    def row(i, carry):
        for q in range(KC // 16):
            sl = pl.ds(q * 16, 16)
            si = src_v[i, sl]
            di = dst_v[i, sl]
            dxb[i, sl] = plsc.load_gather(locx, [di]) - plsc.load_gather(locx, [si])
            dyb[i, sl] = plsc.load_gather(locy, [di]) - plsc.load_gather(locy, [si])
        return carry

    lax.fori_loop(0, half, row, 0)
    pltpu.sync_copy(dxb, dx_out.at[s, pl.ds(c * half, half)])
    pltpu.sync_copy(dyb, dy_out.at[s, pl.ds(c * half, half)])


_dloc = functools.partial(
    pl.kernel,
    out_type=[_sds((NT, NCHUNK, KC), F32), _sds((NT, NCHUNK, KC), F32)],
    mesh=_mesh,
    compiler_params=pltpu.CompilerParams(needs_layout_passes=False),
    scratch_types=[
        pltpu.VMEM((10240,), F32),
        pltpu.VMEM((10240,), F32),
        pltpu.VMEM((NCHUNK // NC, KC), jnp.int32),
        pltpu.VMEM((NCHUNK // NC, KC), jnp.int32),
        pltpu.VMEM((NCHUNK // NC, KC), F32),
        pltpu.VMEM((NCHUNK // NC, KC), F32),
    ],
)(_dloc_body)


RFL = 32
NPIECE = NCHUNK // RFL


def _agg(table, acc, srcr2, dstr, wr, c, s, src_v, dst_v, w_v, ra, rb,
         gsem, ssem):
    def scale(buf, j):
        def egroup(g, c2):
            w16 = w_v[j, pl.ds(g * 16, 16)]
            for i in range(16):
                wvec = jnp.broadcast_to(w16[i], (16,))
                e = g * 16 + i
                for q in range(CH // 16):
                    sl = pl.ds(q * 16, 16)
                    buf[e, sl] = buf[e, sl] * wvec
            return c2

        lax.fori_loop(0, KC // 16, egroup, 0)

    def piece(p, carry):
        off = pl.ds(p * RFL, RFL)
        pltpu.sync_copy(srcr2.at[c, s, off], src_v)
        pltpu.sync_copy(dstr.at[s, off], dst_v)
        pltpu.sync_copy(wr.at[s, off], w_v)
        pltpu.async_copy(table.at[src_v.at[0]], ra, gsem)

        def pair(j2, c1):
            j0 = 2 * j2
            j1 = j0 + 1

            @pl.when(j2 > 0)
            def _():
                pltpu.make_async_copy(rb, acc.at[dst_v.at[j0]], ssem).wait()

            pltpu.async_copy(table.at[src_v.at[j1]], rb, gsem)
            pltpu.make_async_copy(table.at[src_v.at[j0]], ra, gsem).wait()
            scale(ra, j0)
            pltpu.async_copy(ra, acc.at[dst_v.at[j0]], ssem, add=True)
            pltpu.make_async_copy(table.at[src_v.at[j1]], rb, gsem).wait()
            scale(rb, j1)
            pltpu.async_copy(rb, acc.at[dst_v.at[j1]], ssem, add=True)
            pltpu.make_async_copy(ra, acc.at[dst_v.at[j0]], ssem).wait()

            @pl.when(j2 < RFL // 2 - 1)
            def _():
                pltpu.async_copy(table.at[src_v.at[j0 + 2]], ra, gsem)

            return c1

        lax.fori_loop(0, RFL // 2, pair, 0)
        pltpu.make_async_copy(rb, acc.at[dst_v.at[RFL - 1]], ssem).wait()
        return carry

    lax.fori_loop(0, NPIECE, piece, 0)


def _block_body(y0r, y1r, y2r, srcr2, dstr, wr, tmid, hout,
                acc, src_v, dst_v, w_v, ra, rb, gsem, ssem):
    c = lax.axis_index("c")
    s = lax.axis_index("s")
    srow = s * RPT
    hbase = c * NP + srow
    pltpu.sync_copy(y1r.at[pl.ds(hbase, RPT)], acc.at[pl.ds(srow, RPT)])
    plsc.subcore_barrier()
    _agg(y2r, acc, srcr2, dstr, wr, c, s, src_v, dst_v, w_v, ra, rb,
         gsem, ssem)
    plsc.subcore_barrier()
    pltpu.sync_copy(acc.at[pl.ds(srow, RPT)], tmid.at[pl.ds(hbase, RPT)])
    pltpu.sync_copy(y0r.at[pl.ds(hbase, RPT)], acc.at[pl.ds(srow, RPT)])
    plsc.subcore_barrier()
    _agg(tmid, acc, srcr2, dstr, wr, c, s, src_v, dst_v, w_v, ra, rb,
         gsem, ssem)
    plsc.subcore_barrier()
    piece = KC
    for t in range(RPT // KC):
        pltpu.sync_copy(acc.at[pl.ds(srow + t * piece, piece)], ra)

        def rrow(i, carry):
            for q in range(CH // 16):
                sl = pl.ds(q * 16, 16)
                ra[i, sl] = jnp.maximum(ra[i, sl], 0.0)
            return carry

        lax.fori_loop(0, piece, rrow, 0)
        pltpu.sync_copy(ra, hout.at[pl.ds(hbase + t * piece, piece)])


_blockk = functools.partial(
    pl.kernel,
    out_type=[_sds((NC * NP, CH), F32), _sds((NC * NP, CH), F32)],
    mesh=_mesh,
    compiler_params=pltpu.CompilerParams(needs_layout_passes=False),
    scratch_types=[
        pltpu.VMEM_SHARED((NP, CH), F32),
        pltpu.VMEM((RFL, KC), jnp.int32),
        pltpu.VMEM((RFL, KC), jnp.int32),
        pltpu.VMEM((RFL, KC), F32),
        pltpu.VMEM((KC, CH), F32),
        pltpu.VMEM((KC, CH), F32),
        pltpu.SemaphoreType.DMA,
        pltpu.SemaphoreType.DMA,
    ],
)(_block_body)


def _mlp_block(x_ref, w1_ref, b1_ref, w2_ref, b2_ref, o_ref):
    h = jnp.tanh(
        jnp.dot(x_ref[...], w1_ref[...], preferred_element_type=F32)
        + b1_ref[...])
    o_ref[...] = jnp.dot(h, w2_ref[...], preferred_element_type=F32) + b2_ref[...]


def _mlp(ker_in, Wk1, bk1, Wk2, bk2):
    blk = 8192
    grid = EPAD // blk
    return pl.pallas_call(
        _mlp_block,
        grid=(grid,),
        in_specs=[
            pl.BlockSpec((blk, 4), lambda i: (i, 0)),
            pl.BlockSpec((4, 16), lambda i: (0, 0)),
            pl.BlockSpec((1, 16), lambda i: (0, 0)),
            pl.BlockSpec((16, 1), lambda i: (0, 0)),
            pl.BlockSpec((1, 1), lambda i: (0, 0)),
        ],
        out_specs=pl.BlockSpec((blk, 1), lambda i: (i, 0)),
        out_shape=_sds((EPAD, 1), F32),
    )(ker_in, Wk1, bk1, Wk2, bk2)


NB = 512
NPB = NP // NB


def _proj_block(x_ref, xn_ref, wf_ref, bf_ref, wc_ref, o0_ref, o1_ref, o2_ref):
    wc = wc_ref[...]
    wf = wf_ref[...]
    bf = bf_ref[...]
    xn = xn_ref[...]
    outs = [o0_ref, o1_ref, o2_ref]
    for k in range(3):
        wk = wc[18 * k:18 * k + 18]
        m = jnp.dot(wf, wk[0:8], preferred_element_type=F32) + wk[16:18]
        c0 = jnp.dot(bf, wk[0:8], preferred_element_type=F32)
        parts = [
            jnp.dot(x_ref[srem], m, preferred_element_type=F32) + c0
            for srem in range(8)
        ]
        wn_big = jnp.concatenate([wk[8:16]] * 8, axis=1)
        outs[k][...] = (jnp.concatenate(parts, axis=1)
                        + jnp.dot(xn, wn_big, preferred_element_type=F32))


def _proj(x16, node_emb_p, W_feat, b_feat, W_conv1):
    return pl.pallas_call(
        _proj_block,
        grid=(NC, NPB),
        in_specs=[
            pl.BlockSpec((8, NB, 2), lambda c, i: (c, i, 0)),
            pl.BlockSpec((NB, 8), lambda c, i: (i, 0)),
            pl.BlockSpec((2, 8), lambda c, i: (0, 0)),
            pl.BlockSpec((1, 8), lambda c, i: (0, 0)),
            pl.BlockSpec((54, 16), lambda c, i: (0, 0)),
        ],
        out_specs=[
            pl.BlockSpec((NB, CH), lambda c, i: (c * NPB + i, 0))
            for _ in range(3)],
        out_shape=[_sds((NC * NP, CH), F32) for _ in range(3)],
    )(x16, node_emb_p, W_feat, b_feat, W_conv1)


def _mid_block(h_ref, wc2_ref, o0_ref, o1_ref, o2_ref):
    wc2 = wc2_ref[...]
    h = h_ref[...]
    outs = [o0_ref, o1_ref, o2_ref]
    for k in range(3):
        vk = wc2[16 * k:16 * k + 16]
        bd = jnp.concatenate(
            [jnp.pad(vk, ((0, 0), (16 * srem, 112 - 16 * srem)))
             for srem in range(8)], axis=0)
        outs[k][...] = jnp.dot(h, bd, preferred_element_type=F32)


def _mid(h1, W_conv2):
    return pl.pallas_call(
        _mid_block,
        grid=(NC * NPB,),
        in_specs=[
            pl.BlockSpec((NB, CH), lambda i: (i, 0)),
            pl.BlockSpec((48, 16), lambda i: (0, 0)),
        ],
        out_specs=[
            pl.BlockSpec((NB, CH), lambda i: (i, 0)) for _ in range(3)],
        out_shape=[_sds((NC * NP, CH), F32) for _ in range(3)],
    )(h1, W_conv2)


def _final_block(x_ref, wo_ref, bo_ref, o_ref):
    wo = wo_ref[...]
    wbig = jnp.concatenate([wo] * 8, axis=0) * (1.0 / 8.0)
    o_ref[...] = (
        jnp.dot(x_ref[...], wbig, preferred_element_type=F32) + bo_ref[...])


def _final(h2, W_out, b_out):
    blk = 2048
    grid = (NC * NP) // blk
    return pl.pallas_call(
        _final_block,
        grid=(grid,),
        in_specs=[
            pl.BlockSpec((blk, CH), lambda i: (i, 0)),
            pl.BlockSpec((16, 8), lambda i: (0, 0)),
            pl.BlockSpec((1, 8), lambda i: (0, 0)),
        ],
        out_specs=pl.BlockSpec((blk, 8), lambda i: (i, 0)),
        out_shape=_sds((NC * NP, 8), F32),
    )(h2, W_out, b_out)


def kernel(inputs, loc_info, sparse_idx, geodesic, angle_ratio,
           node_embeddings, W_feat, b_feat, Wk1, bk1, Wk2, bk2,
           W_conv1, W_conv2, W_out, b_out):
    dst = sparse_idx[0]
    src = sparse_idx[1]
    pad = EPAD - E
    zi = jnp.zeros((pad,), jnp.int32)
    src_p = jnp.concatenate([src, zi]).reshape(NT, NCHUNK, KC)
    dst_p = jnp.concatenate([dst, zi]).reshape(NT, NCHUNK, KC)
    srcr2 = jnp.stack([src_p, src_p + NP])

    locT = jnp.concatenate([loc_info.T, jnp.zeros((2, 10240 - N), F32)], axis=1)
    dx, dy = _dloc(locT, src_p, dst_p)
    zf = jnp.zeros((pad,), F32)
    ker_in = jnp.stack([
        dx.reshape(-1), dy.reshape(-1),
        jnp.concatenate([geodesic, zf]),
        jnp.concatenate([angle_ratio, zf])], axis=-1)
    w_mlp = _mlp(ker_in, Wk1, bk1.reshape(1, 16), Wk2, bk2.reshape(1, 1))
    wr = jnp.concatenate([w_mlp[:E, 0], zf]).reshape(NT, NCHUNK, KC)

    x16 = jnp.pad(inputs.reshape(16, N, 2), ((0, 0), (0, NP - N), (0, 0)))
    node_emb_p = jnp.pad(node_embeddings, ((0, NP - N), (0, 0)))
    y0, y1, y2 = _proj(x16, node_emb_p, W_feat, b_feat.reshape(1, 8), W_conv1)
    _, h1 = _blockk(y0, y1, y2, srcr2, dst_p, wr)
    z0, z1, z2 = _mid(h1, W_conv2)
    _, h2 = _blockk(z0, z1, z2, srcr2, dst_p, wr)
    o = _final(h2, W_out, b_out.reshape(1, 8))
    return o.reshape(2, NP, 8)[:, :N].transpose(2, 0, 1)
